# baseline (speedup 1.0000x reference)
import jax
import jax.numpy as jnp
from jax import lax
from jax.experimental import pallas as pl
from jax.experimental.pallas import tpu as pltpu
from jax.experimental.pallas import tpu_sc as plsc

_NUM_FIXED = 256
_TOTAL_VT = 1024
_TOKEN_DIM = 4096
_BATCH = 16

_NC = 2
_NS = 16
_NW = _NC * _NS

_B_FLAT = _BATCH * _TOTAL_VT
_B_PER_W = _B_FLAT // _NW
_CH = 8
_NCHUNK = _B_PER_W // _CH
_NBUF = 3
_NFULL = _NCHUNK // _NBUF
_REM = _NCHUNK - _NFULL * _NBUF


_MBLK = 128
_MGRID = _TOTAL_VT // _MBLK
_MFIX = _NUM_FIXED // _MBLK


def _merge_body(emb_ref, train_ref, out_ref):
    g = pl.program_id(0)

    @pl.when(g < _MFIX)
    def _():
        out_ref[...] = emb_ref[...]

    @pl.when(g >= _MFIX)
    def _():
        out_ref[...] = train_ref[...]


def _gather_body(tbl_hbm, idx_hbm, out_hbm, idx_v, *rest):
    bufs = rest[:_NBUF]
    gsems = rest[_NBUF:2 * _NBUF]
    wsems = rest[2 * _NBUF:3 * _NBUF]

    c = lax.axis_index("c")
    s = lax.axis_index("s")
    wid = s * _NC + c
    out0 = wid * _B_PER_W

    pltpu.sync_copy(idx_hbm.at[pl.ds(wid * _NCHUNK, _NCHUNK)], idx_v)

    def start_gather(chunk, b):
        pltpu.async_copy(tbl_hbm.at[idx_v.at[chunk]], bufs[b], gsems[b])

    def wait_gather(b):
        pltpu.make_async_copy(tbl_hbm.at[pl.ds(0, _CH)], bufs[b], gsems[b]).wait()

    def start_write(chunk, b):
        pltpu.async_copy(bufs[b], out_hbm.at[pl.ds(out0 + chunk * _CH, _CH)],
                         wsems[b])

    def wait_write(chunk, b):
        pltpu.make_async_copy(bufs[b],
                              out_hbm.at[pl.ds(out0 + chunk * _CH, _CH)],
                              wsems[b]).wait()

    for b in range(_NBUF):
        start_gather(b, b)

    @pl.loop(0, (_NFULL - 1) * _NBUF, step=_NBUF)
    def _(base):
        for b in range(_NBUF):
            wait_gather(b)
            start_write(base + b, b)
        for b in range(_NBUF):
            wait_write(base + b, b)
            start_gather(base + _NBUF + b, b)

    last = (_NFULL - 1) * _NBUF
    for b in range(_NBUF):
        wait_gather(b)
        start_write(last + b, b)
    for r in range(_REM):
        wait_write(last + r, r)
        start_gather(_NFULL * _NBUF + r, r)
    for r in range(_REM):
        wait_gather(r)
        start_write(_NFULL * _NBUF + r, r)
    for b in range(_REM, _NBUF):
        wait_write(last + b, b)
    for r in range(_REM):
        wait_write(_NFULL * _NBUF + r, r)


@jax.jit
def _run(indices_2d, embeddings_weight, trainable_weight):
    mesh = plsc.VectorSubcoreMesh(core_axis_name="c", subcore_axis_name="s")
    merged = pl.pallas_call(
        _merge_body,
        out_shape=jax.ShapeDtypeStruct((_TOTAL_VT, _TOKEN_DIM), jnp.float32),
        grid=(_MGRID,),
        in_specs=[
            pl.BlockSpec((_MBLK, _TOKEN_DIM),
                         lambda g: (jnp.minimum(g, _MFIX - 1), 0)),
            pl.BlockSpec((_MBLK, _TOKEN_DIM),
                         lambda g: (jnp.maximum(g, _MFIX) - _MFIX, 0)),
        ],
        out_specs=pl.BlockSpec((_MBLK, _TOKEN_DIM), lambda g: (g, 0)),
    )(embeddings_weight, trainable_weight)
    gather = pl.kernel(
        _gather_body,
        out_type=jax.ShapeDtypeStruct((_B_FLAT, _TOKEN_DIM), jnp.float32),
        mesh=mesh,
        scratch_types=(
            [pltpu.VMEM((_NCHUNK, _CH), jnp.int32)]
            + [pltpu.VMEM((_CH, _TOKEN_DIM), jnp.float32) for _ in range(_NBUF)]
            + [pltpu.SemaphoreType.DMA for _ in range(2 * _NBUF)]
        ),
    )
    return gather(merged, indices_2d)


def kernel(indices, embeddings_weight, trainable_weight):
    idx_2d = indices.astype(jnp.int32).reshape(_B_FLAT // _CH, _CH)
    out = _run(idx_2d, embeddings_weight, trainable_weight)
    return out.reshape(_BATCH, _TOTAL_VT, _TOKEN_DIM)
